# Initial kernel scaffold; baseline (speedup 1.0000x reference)
import jax
import jax.numpy as jnp
from jax.experimental import pallas as pl


def kernel(x, edge_index, W1, b1, W2, b2):
    raise NotImplementedError("write your pallas kernel here")


# speedup vs baseline: 6.7072x; 6.7072x over previous
import functools

import jax
import jax.numpy as jnp
from jax import lax
from jax.experimental import pallas as pl
from jax.experimental.pallas import tpu as pltpu
from jax.experimental.pallas import tpu_sc as plsc

N = 10000
E = 320000
D_IN = 128
D_HID = 256
D_OUT = 128

NC = 2
NS = 16
NW = NC * NS

PADN = 10240
STRIP = PADN // NS
CH = 128

EP = 323584
EPW = EP // NW
T_AGG = EPW // CH

DEG_EPW = E // NW
T_DEG = DEG_EPW // CH
DEG_TAIL = DEG_EPW - T_DEG * CH

_mesh = plsc.VectorSubcoreMesh(core_axis_name="c", subcore_axis_name="s")
_sc_params = pltpu.CompilerParams(
    use_tc_tiling_on_sc=False, needs_layout_passes=False)


@functools.partial(
    pl.kernel,
    mesh=_mesh,
    compiler_params=_sc_params,
    out_type=jax.ShapeDtypeStruct((NC, 2, PADN), jnp.float32),
    scratch_types=[
        pltpu.VMEM((PADN,), jnp.float32),
        pltpu.VMEM((PADN,), jnp.float32),
        pltpu.VMEM((CH,), jnp.int32),
        pltpu.VMEM((CH,), jnp.int32),
        pltpu.VMEM((DEG_TAIL,), jnp.int32),
        pltpu.VMEM((DEG_TAIL,), jnp.int32),
        pltpu.VMEM((STRIP,), jnp.float32),
        pltpu.VMEM((STRIP,), jnp.float32),
        pltpu.VMEM_SHARED((NS, 2, PADN), jnp.float32),
    ],
)
def _deg_sc(src_hbm, dst_hbm, out_hbm, dsrc, ddst, sbuf, dbuf, stail, dtail,
            abuf, ibuf, stg):
    c = lax.axis_index("c")
    s = lax.axis_index("s")
    wid = c * NS + s
    zero16 = jnp.zeros((16,), jnp.float32)
    one16 = jnp.ones((16,), jnp.float32)

    def zbody(i, _):
        dsrc[pl.ds(i * 16, 16)] = zero16
        ddst[pl.ds(i * 16, 16)] = zero16
        return 0

    lax.fori_loop(0, PADN // 16, zbody, 0)

    def ebody(t, _):
        base = pl.multiple_of(wid * DEG_EPW + t * CH, 8)
        pltpu.sync_copy(src_hbm.at[pl.ds(base, CH)], sbuf)
        pltpu.sync_copy(dst_hbm.at[pl.ds(base, CH)], dbuf)
        for g in range(CH // 16):
            plsc.addupdate_scatter(dsrc, [sbuf[pl.ds(g * 16, 16)]], one16)
            plsc.addupdate_scatter(ddst, [dbuf[pl.ds(g * 16, 16)]], one16)
        return 0

    lax.fori_loop(0, T_DEG, ebody, 0)

    tbase = pl.multiple_of(wid * DEG_EPW + T_DEG * CH, 8)
    pltpu.sync_copy(src_hbm.at[pl.ds(tbase, DEG_TAIL)], stail)
    pltpu.sync_copy(dst_hbm.at[pl.ds(tbase, DEG_TAIL)], dtail)
    plsc.addupdate_scatter(dsrc, [stail[...]], one16)
    plsc.addupdate_scatter(ddst, [dtail[...]], one16)

    pltpu.sync_copy(dsrc, stg.at[s, 0])
    pltpu.sync_copy(ddst, stg.at[s, 1])
    plsc.subcore_barrier()

    off = pl.multiple_of(s * STRIP, 8)
    for which in (0, 1):
        pltpu.sync_copy(stg.at[0, which, pl.ds(off, STRIP)], abuf)

        def tbody(t, _):
            pltpu.sync_copy(stg.at[t, which, pl.ds(off, STRIP)], ibuf)

            def kbody(k, _):
                sl = pl.ds(k * 16, 16)
                abuf[sl] = abuf[sl] + ibuf[sl]
                return 0

            lax.fori_loop(0, STRIP // 16, kbody, 0)
            return 0

        lax.fori_loop(1, NS, tbody, 0)
        pltpu.sync_copy(abuf, out_hbm.at[c, which, pl.ds(off, STRIP)])


@functools.partial(
    pl.kernel,
    mesh=_mesh,
    compiler_params=_sc_params,
    out_type=jax.ShapeDtypeStruct((NC, PADN, D_IN), jnp.float32),
    scratch_types=[
        pltpu.VMEM((CH,), jnp.int32),
        pltpu.VMEM((CH,), jnp.int32),
        pltpu.VMEM((CH, D_IN), jnp.float32),
        pltpu.VMEM((CH, D_IN), jnp.float32),
        pltpu.VMEM_SHARED((PADN, D_IN), jnp.float32),
        pltpu.SemaphoreType.DMA,
    ],
)
def _agg_sc(src_hbm, dst_hbm, feat_hbm, out_hbm, sidx, didx, rows, zbuf, acc, sem):
    c = lax.axis_index("c")
    s = lax.axis_index("s")
    wid = c * NS + s
    zero16 = jnp.zeros((16,), jnp.float32)

    def zbody(i, _):
        r = i // (D_IN // 16)
        g = i % (D_IN // 16)
        zbuf[r, pl.ds(g * 16, 16)] = zero16
        return 0

    lax.fori_loop(0, CH * D_IN // 16, zbody, 0)

    off = s * STRIP
    for j in range(STRIP // CH):
        pltpu.sync_copy(zbuf, acc.at[pl.ds(off + j * CH, CH)])
    plsc.subcore_barrier()

    def ebody(t, _):
        base = pl.multiple_of(wid * EPW + t * CH, 8)
        pltpu.sync_copy(src_hbm.at[pl.ds(base, CH)], sidx)
        pltpu.sync_copy(dst_hbm.at[pl.ds(base, CH)], didx)
        pltpu.async_copy(feat_hbm.at[sidx], rows, sem).wait()
        pltpu.sync_copy(rows, acc.at[didx], add=True)
        return 0

    lax.fori_loop(0, T_AGG, ebody, 0)
    plsc.subcore_barrier()

    pltpu.sync_copy(acc.at[pl.ds(off, STRIP)], out_hbm.at[c, pl.ds(off, STRIP)])


def _xs_body(x_ref, deg_ref, o_ref):
    d = deg_ref[...]
    ns = lax.rsqrt(d[0, 0] + d[1, 0] + 1.0)
    o_ref[...] = x_ref[...] * ns


def _mid_body(p_ref, xs_ref, deg_ref, w1_ref, b1_ref, w2_ref, o_ref):
    d = deg_ref[...]
    ns = lax.rsqrt(d[0, 0] + d[1, 0] + 1.0)
    nd = lax.rsqrt(d[0, 1] + d[1, 1] + 1.0)
    agg = (p_ref[0] + p_ref[1] + xs_ref[...]) * nd
    h1 = jnp.dot(agg, w1_ref[...], preferred_element_type=jnp.float32)
    h1 = jnp.maximum(h1 + b1_ref[...], 0.0)
    o_ref[...] = jnp.dot(h1 * ns, w2_ref[...], preferred_element_type=jnp.float32)


def _fin_body(p_ref, z_ref, deg_ref, b2_ref, o_ref):
    d = deg_ref[...]
    nd = lax.rsqrt(d[0, 1] + d[1, 1] + 1.0)
    agg = (p_ref[0] + p_ref[1] + z_ref[...]) * nd
    o_ref[...] = jnp.maximum(agg + b2_ref[...], 0.0)


_RB = 2000
_G = N // _RB

_deg_spec = pl.BlockSpec((NC, 2, _RB, 1), lambda i: (0, 0, i, 0))
_p_spec = pl.BlockSpec((NC, _RB, D_IN), lambda i: (0, i, 0))
_row_spec = pl.BlockSpec((_RB, D_IN), lambda i: (i, 0))

_xs_tc = pl.pallas_call(
    _xs_body,
    grid=(_G,),
    in_specs=[_row_spec, _deg_spec],
    out_specs=_row_spec,
    out_shape=jax.ShapeDtypeStruct((N, D_IN), jnp.float32),
)

_mid_tc = pl.pallas_call(
    _mid_body,
    grid=(_G,),
    in_specs=[
        _p_spec,
        _row_spec,
        _deg_spec,
        pl.BlockSpec((D_IN, D_HID), lambda i: (0, 0)),
        pl.BlockSpec((1, D_HID), lambda i: (0, 0)),
        pl.BlockSpec((D_HID, D_OUT), lambda i: (0, 0)),
    ],
    out_specs=_row_spec,
    out_shape=jax.ShapeDtypeStruct((N, D_OUT), jnp.float32),
)

_fin_tc = pl.pallas_call(
    _fin_body,
    grid=(_G,),
    in_specs=[
        _p_spec,
        _row_spec,
        _deg_spec,
        pl.BlockSpec((1, D_OUT), lambda i: (0, 0)),
    ],
    out_specs=_row_spec,
    out_shape=jax.ShapeDtypeStruct((N, D_OUT), jnp.float32),
)


def kernel(x, edge_index, W1, b1, W2, b2):
    npad = EP - E
    pad = jnp.stack([
        jnp.zeros((npad,), jnp.int32),
        jnp.full((npad,), N, jnp.int32),
    ])
    ep = jnp.concatenate([edge_index, pad], axis=1)
    srcp = ep[0]
    dstp = ep[1]

    degs = _deg_sc(srcp, dstp)
    degs4 = degs[:, :, :, None]

    xs = _xs_tc(x, degs4)
    p1 = _agg_sc(srcp, dstp, xs)
    z = _mid_tc(p1, xs, degs4, W1, b1.reshape(1, D_HID), W2)
    p2 = _agg_sc(srcp, dstp, z)
    return _fin_tc(p2, z, degs4, b2.reshape(1, D_OUT))
